# baseline (speedup 1.0000x reference)
import functools

import jax
import jax.numpy as jnp
from jax import lax
from jax.experimental import pallas as pl
from jax.experimental.pallas import tpu as pltpu
from jax.experimental.pallas import tpu_sc as plsc

N = 512
BI = 64
N_ELEM = 100


def _radii_gather(radii_table, numbers):
    info = plsc.get_sparse_core_info()
    nw = info.num_cores * info.num_subcores
    per = N // nw
    mesh = plsc.VectorSubcoreMesh(core_axis_name="c", subcore_axis_name="s")

    @functools.partial(
        pl.kernel, mesh=mesh,
        out_type=jax.ShapeDtypeStruct((N,), jnp.float32),
        scratch_types=[
            pltpu.VMEM((per,), jnp.int32),
            pltpu.VMEM((per,), jnp.float32),
            pltpu.SemaphoreType.DMA,
        ],
    )
    def gather_k(table_hbm, idx_hbm, out_hbm, idx_v, out_v, sem):
        wid = lax.axis_index("s") * info.num_cores + lax.axis_index("c")
        base = wid * per
        pltpu.sync_copy(idx_hbm.at[pl.ds(base, per)], idx_v)
        pltpu.async_copy(table_hbm.at[idx_v], out_v, sem).wait()
        pltpu.sync_copy(out_v, out_hbm.at[pl.ds(base, per)])

    table_pad = jnp.pad(radii_table.astype(jnp.float32), (0, 128 - N_ELEM))
    return gather_k(table_pad, numbers.astype(jnp.int32))


def _body(pj_ref, sv_ref, rj_ref, pi_ref, ri_ref, out_ref):
    pj = pj_ref[...]
    pj0 = pj[0:1, :]
    pj1 = pj[1:2, :]
    pj2 = pj[2:3, :]
    pi = pi_ref[...]
    pi0 = pi[:, 0:1]
    pi1 = pi[:, 1:2]
    pi2 = pi[:, 2:3]
    dx0 = pj0 - pi0
    dx1 = pj1 - pi1
    dx2 = pj2 - pi2
    rj = rj_ref[...][0:1, :]
    gcut = 2.0 * jnp.max(rj)
    cut = jnp.minimum(ri_ref[...][:, 0:1] + rj, gcut)
    eps = jnp.float32(1e-8)
    for s in range(27):
        d0 = dx0 + sv_ref[s, 0]
        d1 = dx1 + sv_ref[s, 1]
        d2 = dx2 + sv_ref[s, 2]
        rs = jnp.sqrt((d0 * d0 + d1 * d1) + d2 * d2)
        mask = (rs <= cut) & (rs > eps)
        out_ref[s, 0, :, :] = jnp.where(mask, d0, 0.0)
        out_ref[s, 1, :, :] = jnp.where(mask, d1, 0.0)
        out_ref[s, 2, :, :] = jnp.where(mask, d2, 0.0)


def _field(PJ, SV, RJ, PI, RI):
    return pl.pallas_call(
        _body,
        grid=(N // BI,),
        in_specs=[
            pl.BlockSpec((8, N), lambda i: (0, 0)),
            pl.BlockSpec((32, 128), lambda i: (0, 0)),
            pl.BlockSpec((8, N), lambda i: (0, 0)),
            pl.BlockSpec((BI, 128), lambda i: (i, 0)),
            pl.BlockSpec((BI, 128), lambda i: (i, 0)),
        ],
        out_specs=pl.BlockSpec((27, 3, BI, N), lambda i: (0, 0, i, 0)),
        out_shape=jax.ShapeDtypeStruct((27, 3, N, N), jnp.float32),
    )(PJ, SV, RJ, PI, RI)


def kernel(positions, cell, radii_table, numbers):
    positions = positions.astype(jnp.float32)
    s = jnp.arange(-1, 2, dtype=positions.dtype)
    g = jnp.meshgrid(s, s, s, indexing="ij")
    shifts = jnp.stack(g, axis=-1).reshape(-1, 3)
    shift_vecs = shifts @ cell

    radii = _radii_gather(radii_table, numbers)

    PJ = jnp.pad(positions.T, ((0, 5), (0, 0)))
    SV = jnp.pad(shift_vecs, ((0, 5), (0, 125)))
    RJ = jnp.broadcast_to(radii[None, :], (8, N))
    PI = jnp.pad(positions, ((0, 0), (0, 125)))
    RI = jnp.broadcast_to(radii[:, None], (N, 128))

    out = _field(PJ, SV, RJ, PI, RI)
    return jnp.transpose(out, (2, 3, 0, 1))


# speedup vs baseline: 1.0215x; 1.0215x over previous
import functools

import jax
import jax.numpy as jnp
from jax import lax
from jax.experimental import pallas as pl
from jax.experimental.pallas import tpu as pltpu
from jax.experimental.pallas import tpu_sc as plsc

N = 512
BI = 64
N_ELEM = 100


def _radii_gather(radii_table, numbers):
    info = plsc.get_sparse_core_info()
    nw = info.num_cores * info.num_subcores
    per = N // nw
    mesh = plsc.VectorSubcoreMesh(core_axis_name="c", subcore_axis_name="s")

    @functools.partial(
        pl.kernel, mesh=mesh,
        out_type=jax.ShapeDtypeStruct((N,), jnp.float32),
        scratch_types=[
            pltpu.VMEM((per,), jnp.int32),
            pltpu.VMEM((per,), jnp.float32),
            pltpu.SemaphoreType.DMA,
        ],
    )
    def gather_k(table_hbm, idx_hbm, out_hbm, idx_v, out_v, sem):
        wid = lax.axis_index("s") * info.num_cores + lax.axis_index("c")
        base = wid * per
        pltpu.sync_copy(idx_hbm.at[pl.ds(base, per)], idx_v)
        pltpu.async_copy(table_hbm.at[idx_v], out_v, sem).wait()
        pltpu.sync_copy(out_v, out_hbm.at[pl.ds(base, per)])

    table_pad = jnp.pad(radii_table.astype(jnp.float32), (0, 128 - N_ELEM))
    return gather_k(table_pad, numbers.astype(jnp.int32))


def _body(pj_ref, sv_ref, rj_ref, pi_ref, ri_ref, out_ref):
    pj = pj_ref[...]
    pj0 = pj[0:1, :]
    pj1 = pj[1:2, :]
    pj2 = pj[2:3, :]
    pi = pi_ref[...]
    pi0 = pi[:, 0:1]
    pi1 = pi[:, 1:2]
    pi2 = pi[:, 2:3]
    dx0 = pj0 - pi0
    dx1 = pj1 - pi1
    dx2 = pj2 - pi2
    rj = rj_ref[...]
    gcut = 2.0 * jnp.max(rj)
    cut = jnp.minimum(ri_ref[...] + rj, gcut)
    eps = jnp.float32(1e-8)
    for s in range(27):
        d0 = dx0 + sv_ref[s, 0]
        d1 = dx1 + sv_ref[s, 1]
        d2 = dx2 + sv_ref[s, 2]
        rs = jnp.sqrt((d0 * d0 + d1 * d1) + d2 * d2)
        mask = (rs <= cut) & (rs > eps)
        out_ref[s, 0, :, :] = jnp.where(mask, d0, 0.0)
        out_ref[s, 1, :, :] = jnp.where(mask, d1, 0.0)
        out_ref[s, 2, :, :] = jnp.where(mask, d2, 0.0)


def _field(PJ, SV, RJ, PI, RI):
    return pl.pallas_call(
        _body,
        grid=(N // BI,),
        in_specs=[
            pl.BlockSpec((8, N), lambda i: (0, 0)),
            pl.BlockSpec((32, 128), lambda i: (0, 0)),
            pl.BlockSpec((1, N), lambda i: (0, 0)),
            pl.BlockSpec((BI, 3), lambda i: (i, 0)),
            pl.BlockSpec((BI, 1), lambda i: (i, 0)),
        ],
        out_specs=pl.BlockSpec((27, 3, BI, N), lambda i: (0, 0, i, 0)),
        out_shape=jax.ShapeDtypeStruct((27, 3, N, N), jnp.float32),
    )(PJ, SV, RJ, PI, RI)


def kernel(positions, cell, radii_table, numbers):
    positions = positions.astype(jnp.float32)
    s = jnp.arange(-1, 2, dtype=positions.dtype)
    g = jnp.meshgrid(s, s, s, indexing="ij")
    shifts = jnp.stack(g, axis=-1).reshape(-1, 3)
    shift_vecs = shifts @ cell

    radii = _radii_gather(radii_table, numbers)

    PJ = jnp.pad(positions.T, ((0, 5), (0, 0)))
    SV = jnp.pad(shift_vecs, ((0, 5), (0, 125)))
    RJ = radii.reshape(1, N)
    PI = positions
    RI = radii.reshape(N, 1)

    out = _field(PJ, SV, RJ, PI, RI)
    return jnp.transpose(out, (2, 3, 0, 1))
